# baseline (speedup 1.0000x reference)
import functools

import jax
import jax.numpy as jnp
from jax import lax
from jax.experimental import pallas as pl
from jax.experimental.pallas import tpu as pltpu
from jax.experimental.pallas import tpu_sc as plsc

VOCAB = 1000000
EMBED = 64
OUT = 100000
B = 1024
L = 200

NC = 2
NS = 16
NW = NC * NS
BPW = B // NW
CH1, CH2 = 128, 72
PAIRS = VOCAB // 2
CW = 512
NCHUNK = 1953
TPW = 61


def _repack_body(tt_hbm, out_hbm, in_a, in_b, out_a, out_b,
                 s_ia, s_ib, s_oa, s_ob):
    wid = lax.axis_index("s") * NC + lax.axis_index("c")
    lanes = lax.iota(jnp.int32, 16)
    ins = (in_a, in_b)
    outs = (out_a, out_b)
    isems = (s_ia, s_ib)
    osems = (s_oa, s_ob)

    def transpose_chunk(src, dst, nrows):
        def row_body(qq, _):
            cv = lanes * 0 + 2 * qq
            for u in range(2):
                for eg in range(4):
                    vals = plsc.load_gather(
                        src, [lanes + 16 * eg, cv + u])
                    dst[qq, pl.ds(u * EMBED + 16 * eg, 16)] = vals
            return 0
        lax.fori_loop(0, nrows, row_body, 0)

    h0 = pltpu.async_copy(
        tt_hbm.at[:, pl.ds(wid * CW, CW)], in_a, s_ia)
    handles_in = [h0]
    handles_out = [None, None]
    for t in range(TPW):
        buf = t % 2
        handles_in[t].wait()
        if t + 1 < TPW:
            c_next = ((t + 1) * NW + wid) * CW
            handles_in.append(pltpu.async_copy(
                tt_hbm.at[:, pl.ds(c_next, CW)], ins[(t + 1) % 2],
                isems[(t + 1) % 2]))
        if handles_out[buf] is not None:
            handles_out[buf].wait()
        transpose_chunk(ins[buf], outs[buf], CW // 2)
        kk = t * NW + wid
        handles_out[buf] = pltpu.async_copy(
            outs[buf], out_hbm.at[pl.ds(kk * (CW // 2), CW // 2)],
            osems[buf])
    handles_out[0].wait()
    handles_out[1].wait()

    @pl.when(wid == 0)
    def _():
        kk = NCHUNK - 1
        pltpu.sync_copy(tt_hbm.at[:, pl.ds(kk * CW, CW)], in_a)
        transpose_chunk(in_a, out_a, CW // 2)
        pltpu.sync_copy(out_a, out_hbm.at[pl.ds(kk * (CW // 2), CW // 2)])


_repack = functools.partial(
    pl.kernel,
    mesh=plsc.VectorSubcoreMesh(core_axis_name="c", subcore_axis_name="s"),
    out_type=jax.ShapeDtypeStruct((PAIRS, 2 * EMBED), jnp.float32),
    scratch_types=[
        pltpu.VMEM((EMBED, CW), jnp.float32),
        pltpu.VMEM((EMBED, CW), jnp.float32),
        pltpu.VMEM((CW // 2, 2 * EMBED), jnp.float32),
        pltpu.VMEM((CW // 2, 2 * EMBED), jnp.float32),
        pltpu.SemaphoreType.DMA,
        pltpu.SemaphoreType.DMA,
        pltpu.SemaphoreType.DMA,
        pltpu.SemaphoreType.DMA,
    ],
    compiler_params=pltpu.CompilerParams(needs_layout_passes=False),
)(_repack_body)


def _sc_pool_body(idx_hbm, ne_hbm, table_hbm, out_hbm, idx_v, ne_v, rows_v,
                  acc_v, sem):
    wid = lax.axis_index("s") * NC + lax.axis_index("c")
    base = wid * BPW
    pltpu.sync_copy(idx_hbm.at[pl.ds(base, BPW)], idx_v)
    pltpu.sync_copy(ne_hbm.at[pl.ds(base, BPW)], ne_v)
    lanes = lax.iota(jnp.int32, 16)
    for i in range(BPW):
        g1 = pltpu.async_copy(
            table_hbm.at[idx_v.at[i, pl.ds(0, CH1)]],
            rows_v.at[pl.ds(0, CH1)], sem)
        g2 = pltpu.async_copy(
            table_hbm.at[idx_v.at[i, pl.ds(CH1, CH2)]],
            rows_v.at[pl.ds(CH1, CH2)], sem)
        g1.wait()
        g2.wait()

        nvec = ne_v[pl.ds((i // 16) * 16, 16)]
        n_e = jnp.sum(jnp.where(lanes == (i % 16), nvec, 0))

        def lo_body(j, carry):
            a0, a1, a2, a3 = carry
            a0 = a0 + rows_v[j, pl.ds(0, 16)]
            a1 = a1 + rows_v[j, pl.ds(16, 16)]
            a2 = a2 + rows_v[j, pl.ds(32, 16)]
            a3 = a3 + rows_v[j, pl.ds(48, 16)]
            return a0, a1, a2, a3

        def hi_body(j, carry):
            a0, a1, a2, a3 = carry
            a0 = a0 + rows_v[j, pl.ds(64, 16)]
            a1 = a1 + rows_v[j, pl.ds(80, 16)]
            a2 = a2 + rows_v[j, pl.ds(96, 16)]
            a3 = a3 + rows_v[j, pl.ds(112, 16)]
            return a0, a1, a2, a3

        z = jnp.zeros((16,), jnp.float32)
        carry = lax.fori_loop(0, n_e, lo_body, (z, z, z, z))
        a0, a1, a2, a3 = lax.fori_loop(n_e, L, hi_body, carry)
        acc_v[i, pl.ds(0, 16)] = a0
        acc_v[i, pl.ds(16, 16)] = a1
        acc_v[i, pl.ds(32, 16)] = a2
        acc_v[i, pl.ds(48, 16)] = a3
    pltpu.sync_copy(acc_v, out_hbm.at[pl.ds(base, BPW)])


_sc_pool = functools.partial(
    pl.kernel,
    mesh=plsc.VectorSubcoreMesh(core_axis_name="c", subcore_axis_name="s"),
    out_type=jax.ShapeDtypeStruct((B, EMBED), jnp.float32),
    scratch_types=[
        pltpu.VMEM((BPW, L), jnp.int32),
        pltpu.VMEM((BPW,), jnp.int32),
        pltpu.VMEM((L, 2 * EMBED), jnp.float32),
        pltpu.VMEM((BPW, EMBED), jnp.float32),
        pltpu.SemaphoreType.DMA,
    ],
    compiler_params=pltpu.CompilerParams(needs_layout_passes=False),
)(_sc_pool_body)


BLK = 2048
NBLK = (OUT + BLK - 1) // BLK
K1 = EMBED + 1


def _mm_body(wt_ref, p_ref, o_ref):
    o_ref[:] = lax.dot_general(
        wt_ref[:], p_ref[:], (((0,), (1,)), ((), ())),
        preferred_element_type=jnp.float32)


def _matmul_t(Wbt, pooled1):
    return pl.pallas_call(
        _mm_body,
        grid=(NBLK,),
        in_specs=[
            pl.BlockSpec((K1, BLK), lambda j: (0, j)),
            pl.BlockSpec((B, K1), lambda j: (0, 0)),
        ],
        out_specs=pl.BlockSpec((BLK, B), lambda j: (j, 0)),
        out_shape=jax.ShapeDtypeStruct((OUT, B), jnp.float32),
    )(Wbt, pooled1)


def kernel(inputs, table, W, b):
    idx = inputs.astype(jnp.int32)
    parity = idx & 1
    order = jnp.argsort(parity, axis=1, stable=True)
    q = jnp.take_along_axis(idx, order, axis=1) >> 1
    ne = (L - parity.sum(axis=1)).astype(jnp.int32)
    table128 = _repack(table.T)
    tail_pack = table[NCHUNK * CW:].reshape(
        (VOCAB - NCHUNK * CW) // 2, 2 * EMBED)
    table128 = lax.dynamic_update_slice(
        table128, tail_pack, (NCHUNK * CW // 2, 0))
    pooled = _sc_pool(q, ne, table128)
    pooled1 = jnp.concatenate([pooled, jnp.ones((B, 1), jnp.float32)], axis=1)
    Wbt = jnp.concatenate([W, b[:, None]], axis=1).T
    out_t = _matmul_t(Wbt, pooled1)
    return out_t.T


# speedup vs baseline: 3.1414x; 3.1414x over previous
import functools

import jax
import jax.numpy as jnp
from jax import lax
from jax.experimental import pallas as pl
from jax.experimental.pallas import tpu as pltpu
from jax.experimental.pallas import tpu_sc as plsc

VOCAB = 1000000
EMBED = 64
OUT = 100000
B = 1024
L = 200

NC = 2
NS = 16
NW = NC * NS
BPW = B // NW
NG = L // 16
REM = L - NG * 16


def _sc_pool_body(idx_hbm, table_hbm, out_hbm, idx_v, rows_v, acc_v, sem):
    wid = lax.axis_index("s") * NC + lax.axis_index("c")
    base = wid * BPW
    pltpu.sync_copy(idx_hbm.at[pl.ds(base, BPW)], idx_v)
    lanes = lax.iota(jnp.int32, 16)

    def extract(vec, l):
        return jnp.sum(jnp.where(lanes == l, vec, 0))

    def row_body(i, carry_unused):
        def fire_group(g, _):
            vec = idx_v[i, pl.ds(g * 16, 16)]
            for l in range(16):
                r = extract(vec, l)
                pltpu.async_copy(
                    table_hbm.at[pl.ds(r, 1)],
                    rows_v.at[pl.ds(g * 16 + l, 1)], sem)
            return 0

        lax.fori_loop(0, NG, fire_group, 0)
        vec = idx_v[i, pl.ds(L - 16, 16)]
        for l in range(16 - REM, 16):
            r = extract(vec, l)
            pltpu.async_copy(
                table_hbm.at[pl.ds(r, 1)],
                rows_v.at[pl.ds(L - 16 + l, 1)], sem)
        pltpu.make_async_copy(table_hbm.at[pl.ds(0, L)], rows_v, sem).wait()

        def acc_body(j, carry):
            a0, a1, a2, a3 = carry
            a0 = a0 + rows_v[j, pl.ds(0, 16)]
            a1 = a1 + rows_v[j, pl.ds(16, 16)]
            a2 = a2 + rows_v[j, pl.ds(32, 16)]
            a3 = a3 + rows_v[j, pl.ds(48, 16)]
            return a0, a1, a2, a3

        z = jnp.zeros((16,), jnp.float32)
        a0, a1, a2, a3 = lax.fori_loop(0, L, acc_body, (z, z, z, z))
        acc_v[i, pl.ds(0, 16)] = a0
        acc_v[i, pl.ds(16, 16)] = a1
        acc_v[i, pl.ds(32, 16)] = a2
        acc_v[i, pl.ds(48, 16)] = a3
        return 0

    lax.fori_loop(0, BPW, row_body, 0)
    pltpu.sync_copy(acc_v, out_hbm.at[pl.ds(base, BPW)])


_sc_pool = functools.partial(
    pl.kernel,
    mesh=plsc.VectorSubcoreMesh(core_axis_name="c", subcore_axis_name="s"),
    out_type=jax.ShapeDtypeStruct((B, EMBED), jnp.float32),
    scratch_types=[
        pltpu.VMEM((BPW, L), jnp.int32),
        pltpu.VMEM((L, EMBED), jnp.float32),
        pltpu.VMEM((BPW, EMBED), jnp.float32),
        pltpu.SemaphoreType.DMA,
    ],
    compiler_params=pltpu.CompilerParams(needs_layout_passes=False),
)(_sc_pool_body)


BLK = 2048
NBLK = (OUT + BLK - 1) // BLK
K1 = EMBED + 1


def _mm_body(wt_ref, p_ref, o_ref):
    o_ref[:] = lax.dot_general(
        wt_ref[:], p_ref[:], (((0,), (1,)), ((), ())),
        preferred_element_type=jnp.float32)


def _matmul_t(Wbt, pooled1):
    return pl.pallas_call(
        _mm_body,
        grid=(NBLK,),
        in_specs=[
            pl.BlockSpec((K1, BLK), lambda j: (0, j)),
            pl.BlockSpec((B, K1), lambda j: (0, 0)),
        ],
        out_specs=pl.BlockSpec((BLK, B), lambda j: (j, 0)),
        out_shape=jax.ShapeDtypeStruct((OUT, B), jnp.float32),
    )(Wbt, pooled1)


def kernel(inputs, table, W, b):
    pooled = _sc_pool(inputs.astype(jnp.int32), table)
    pooled1 = jnp.concatenate([pooled, jnp.ones((B, 1), jnp.float32)], axis=1)
    Wbt = jnp.concatenate([W, b[:, None]], axis=1).T
    out_t = _matmul_t(Wbt, pooled1)
    return out_t.T
